# baseline (speedup 1.0000x reference)
import jax
import jax.numpy as jnp
from jax.experimental import pallas as pl

N = 10000
E = 160000
H = 128
DE = 16
NC = 2
NL = 2
NG = 128
NCLS = 10
G = NC + 1


def _gine(x, ei, ea, ew, gi, W1, b1, W2, b2, gamma, beta, eps, We, be):
    src = ei[0]
    dst = ei[1]
    ea_enc = ea @ We + be
    h = x
    for l in range(NL):
        m = jax.nn.relu(h[src] + ea_enc) * ew[:, None]
        agg = jnp.zeros_like(h).at[dst].add(m)
        z = (1.0 + eps[gi, l]) * h + agg
        z = jax.nn.relu(z @ W1[gi, l] + b1[gi, l])
        z = z @ W2[gi, l] + b2[gi, l]
        z = gamma[gi, l] * z + beta[gi, l]
        z = jax.nn.relu(z)
        h = z + h
    return h


def _pool_mean(h, batch):
    s = jax.ops.segment_sum(h, batch, num_segments=NG)
    c = jax.ops.segment_sum(jnp.ones((h.shape[0], 1), h.dtype), batch, num_segments=NG)
    return s / jnp.maximum(c, 1.0)


def _id_kernel(x_ref, o_ref):
    o_ref[...] = x_ref[...]


def kernel(x, edge_attr, edge_weight, cand_edge_attr, cand_edge_weight, W1, b1, W2, b2, gamma, beta, eps, We, be, Wm, bm, Wf, bf, edge_index, cand_edge_index, batch):
    h = _gine(x, edge_index, edge_attr, edge_weight, 0, W1, b1, W2, b2, gamma, beta, eps, We, be)
    hg = _pool_mean(h, batch)
    hg = jax.nn.relu(hg @ Wm[0] + bm[0])
    h_graphs = [hg]
    for i in range(NC):
        hi = _gine(x, cand_edge_index[i], cand_edge_attr[i], cand_edge_weight[i], i + 1, W1, b1, W2, b2, gamma, beta, eps, We, be)
        hgi = _pool_mean(hi, batch)
        hgi = jax.nn.relu(hgi @ Wm[i + 1] + bm[i + 1])
        h_graphs.append(hgi)
    hG = jnp.stack(h_graphs, axis=0).mean(0)
    out = hG @ Wf + bf
    return pl.pallas_call(
        _id_kernel,
        out_shape=jax.ShapeDtypeStruct(out.shape, out.dtype),
    )(out)


# speedup vs baseline: 1.9817x; 1.9817x over previous
import functools

import jax
import jax.numpy as jnp
from jax import lax
from jax.experimental import pallas as pl
from jax.experimental.pallas import tpu as pltpu
from jax.experimental.pallas import tpu_sc as plsc

N = 10000
E = 160000
H = 128
DE = 16
NC = 2
NL = 2
NG = 128
NCLS = 10
G = NC + 1

CH = 128
EPC = E // 2
QPC = EPC // CH
NSUB = 16
QPW = (QPC + NSUB - 1) // NSUB
RPS = 624


def _make_msg_call(table_rows, offs):
    mesh = plsc.VectorSubcoreMesh(core_axis_name="c", subcore_axis_name="s")

    @functools.partial(
        pl.kernel,
        mesh=mesh,
        out_type=jax.ShapeDtypeStruct((2, G, N, H), jnp.float32),
        scratch_types=[
            pltpu.VMEM((CH,), jnp.int32),
            pltpu.VMEM((CH,), jnp.int32),
            pltpu.VMEM((CH + 16,), jnp.float32),
            pltpu.VMEM((CH, H), jnp.float32),
            pltpu.VMEM((CH, H), jnp.float32),
            pltpu.VMEM_SHARED((N, H), jnp.float32),
            pltpu.SemaphoreType.DMA,
        ],
    )
    def msg(src_hbm, dst_hbm, ew_hbm, eenc_hbm, table_hbm, out_hbm,
            src_v, dst_v, ew_v, rows_v, ea_v, agg_sh, sem):
        cid = lax.axis_index("c")
        sid = lax.axis_index("s")

        rbase = pl.multiple_of(sid * RPS, 8)

        for g in range(G):
            z16 = jnp.zeros((16,), jnp.float32)

            def zrow(i, _):
                for k in range(H // 16):
                    rows_v[i, pl.ds(k * 16, 16)] = z16
                return ()

            lax.fori_loop(0, CH, zrow, ())
            for j in range(4):
                pltpu.sync_copy(rows_v, agg_sh.at[pl.ds(rbase + j * CH, CH)])
            pltpu.sync_copy(rows_v.at[pl.ds(0, 112)],
                            agg_sh.at[pl.ds(rbase + 4 * CH, 112)])

            @pl.when(sid == NSUB - 1)
            def _():
                pltpu.sync_copy(rows_v.at[pl.ds(0, 16)],
                                agg_sh.at[pl.ds(NSUB * RPS, 16)])

            plsc.subcore_barrier()

            def chunk(t, _):
                q = t * NSUB + sid

                @pl.when(q < QPC)
                def _():
                    ebase = pl.multiple_of(g * E + cid * EPC + q * CH, CH)
                    pltpu.sync_copy(src_hbm.at[pl.ds(ebase, CH)], src_v)
                    pltpu.sync_copy(dst_hbm.at[pl.ds(ebase, CH)], dst_v)
                    pltpu.sync_copy(ew_hbm.at[pl.ds(ebase, CH)],
                                    ew_v.at[pl.ds(0, CH)])
                    if offs[g]:
                        for k in range(CH // 16):
                            sl = pl.ds(k * 16, 16)
                            src_v[sl] = src_v[sl] + jnp.int32(offs[g])
                    pltpu.async_copy(table_hbm.at[src_v], rows_v, sem).wait()
                    pltpu.sync_copy(eenc_hbm.at[pl.ds(ebase, CH)], ea_v)

                    def edge(e, _):
                        wj = jnp.full((16,), ew_v[pl.ds(e, 16)][0],
                                      jnp.float32)
                        for k in range(H // 16):
                            sl = pl.ds(k * 16, 16)
                            rows_v[e, sl] = jnp.maximum(
                                rows_v[e, sl] + ea_v[e, sl], 0.0) * wj
                        return ()

                    lax.fori_loop(0, CH, edge, ())
                    pltpu.sync_copy(rows_v, agg_sh.at[dst_v], add=True)

                return ()

            lax.fori_loop(0, QPW, chunk, ())
            plsc.subcore_barrier()
            pltpu.sync_copy(agg_sh.at[pl.ds(rbase, RPS)],
                            out_hbm.at[cid, g, pl.ds(rbase, RPS)])

            @pl.when(sid == NSUB - 1)
            def _():
                pltpu.sync_copy(agg_sh.at[pl.ds(NSUB * RPS, 16)],
                                out_hbm.at[cid, g, pl.ds(NSUB * RPS, 16)])

            plsc.subcore_barrier()

    return msg


def _eenc_body(ea_ref, We_ref, be_ref, o_ref):
    o_ref[...] = (jnp.dot(ea_ref[...], We_ref[...],
                          preferred_element_type=jnp.float32) + be_ref[...])


def _edge_encode(ea_all, We, be):
    RB = 4000
    grid = (3 * E // RB,)
    return pl.pallas_call(
        _eenc_body,
        grid=grid,
        in_specs=[
            pl.BlockSpec((RB, DE), lambda i: (i, 0)),
            pl.BlockSpec((DE, H), lambda i: (0, 0)),
            pl.BlockSpec((1, H), lambda i: (0, 0)),
        ],
        out_specs=pl.BlockSpec((RB, H), lambda i: (i, 0)),
        out_shape=jax.ShapeDtypeStruct((3 * E, H), jnp.float32),
    )(ea_all, We, be.reshape(1, H))


def _dense_body(RB, hin_ref, a0_ref, a1_ref, sc_ref, W1_ref, b1_ref,
                W2_ref, b2_ref, gm_ref, bt_ref, o_ref):
    h = hin_ref[...].reshape(RB, H)
    a = a0_ref[...].reshape(RB, H) + a1_ref[...].reshape(RB, H)
    z = sc_ref[...].reshape(1, H) * h + a
    t = jnp.maximum(jnp.dot(z, W1_ref[...].reshape(H, H),
                            preferred_element_type=jnp.float32)
                    + b1_ref[...].reshape(1, H), 0.0)
    t = (jnp.dot(t, W2_ref[...].reshape(H, H),
                 preferred_element_type=jnp.float32)
         + b2_ref[...].reshape(1, H))
    t = jnp.maximum(gm_ref[...].reshape(1, H) * t
                    + bt_ref[...].reshape(1, H), 0.0)
    o_ref[...] = (t + h).reshape(1, RB, H)


def _dense_layer(l, hin, aggp, W1, b1, W2, b2, gamma, beta, eps):
    RB = 2000
    NB = N // RB
    scl = jnp.broadcast_to((1.0 + eps[:, l])[:, None, None], (G, 1, H))

    def v3(p):
        return p[:, l][:, None, :]

    if l == 0:
        hin_spec = pl.BlockSpec((RB, H), lambda g, i: (i, 0))
    else:
        hin_spec = pl.BlockSpec((1, RB, H), lambda g, i: (g, i, 0))

    vspec = pl.BlockSpec((1, 1, H), lambda g, i: (g, 0, 0))
    return pl.pallas_call(
        functools.partial(_dense_body, RB),
        grid=(G, NB),
        in_specs=[
            hin_spec,
            pl.BlockSpec((1, 1, RB, H), lambda g, i: (0, g, i, 0)),
            pl.BlockSpec((1, 1, RB, H), lambda g, i: (1, g, i, 0)),
            vspec,
            pl.BlockSpec((1, H, H), lambda g, i: (g, 0, 0)),
            vspec,
            pl.BlockSpec((1, H, H), lambda g, i: (g, 0, 0)),
            vspec,
            vspec,
            vspec,
        ],
        out_specs=pl.BlockSpec((1, RB, H), lambda g, i: (g, i, 0)),
        out_shape=jax.ShapeDtypeStruct((G, N, H), jnp.float32),
    )(hin, aggp, aggp, scl, W1[:, l], v3(b1), W2[:, l], v3(b2),
      v3(gamma), v3(beta))


def _pool_body(RB, h_ref, b_ref, s_ref, c_ref):
    g = pl.program_id(0)
    i = pl.program_id(1)
    h = h_ref[...].reshape(RB, H)
    b = b_ref[...]
    oh = (b == lax.broadcasted_iota(jnp.int32, (RB, NG), 1)).astype(jnp.float32)
    sblk = jnp.dot(oh.T, h, preferred_element_type=jnp.float32)

    @pl.when(i == 0)
    def _():
        s_ref[...] = jnp.zeros_like(s_ref)

    s_ref[...] += sblk.reshape(1, NG, H)

    @pl.when((g == 0) & (i == 0))
    def _():
        c_ref[...] = jnp.zeros_like(c_ref)

    @pl.when(g == 0)
    def _():
        cs = jnp.sum(oh, axis=0)
        c_ref[...] += jnp.broadcast_to(cs[:, None], (NG, H))


def _pool(h2, batch2d):
    RB = 2000
    NB = N // RB
    return pl.pallas_call(
        functools.partial(_pool_body, RB),
        grid=(G, NB),
        in_specs=[
            pl.BlockSpec((1, RB, H), lambda g, i: (g, i, 0)),
            pl.BlockSpec((RB, 1), lambda g, i: (i, 0)),
        ],
        out_specs=[
            pl.BlockSpec((1, NG, H), lambda g, i: (g, 0, 0)),
            pl.BlockSpec((NG, H), lambda g, i: (0, 0)),
        ],
        out_shape=[
            jax.ShapeDtypeStruct((G, NG, H), jnp.float32),
            jax.ShapeDtypeStruct((NG, H), jnp.float32),
        ],
    )(h2, batch2d)


def _head_body(s_ref, c_ref, Wm_ref, bm_ref, Wf_ref, bf_ref, o_ref):
    c = jnp.maximum(c_ref[...], 1.0)
    acc = jnp.zeros((NG, H), jnp.float32)
    for g in range(G):
        hg = s_ref[g] / c
        hg = jnp.maximum(jnp.dot(hg, Wm_ref[g],
                                 preferred_element_type=jnp.float32)
                         + bm_ref[g], 0.0)
        acc = acc + hg
    acc = acc * (1.0 / G)
    o_ref[...] = (jnp.dot(acc, Wf_ref[...],
                          preferred_element_type=jnp.float32) + bf_ref[...])


def _head(s, c, Wm, bm, Wf, bf):
    return pl.pallas_call(
        _head_body,
        out_shape=jax.ShapeDtypeStruct((NG, NCLS), jnp.float32),
    )(s, c, Wm, bm[:, None, :], Wf, bf.reshape(1, NCLS))


def kernel(x, edge_attr, edge_weight, cand_edge_attr, cand_edge_weight,
           W1, b1, W2, b2, gamma, beta, eps, We, be, Wm, bm, Wf, bf,
           edge_index, cand_edge_index, batch):
    ei_all = jnp.concatenate([edge_index[None], cand_edge_index], axis=0)
    src_all = ei_all[:, 0, :].reshape(3 * E)
    dst_all = ei_all[:, 1, :].reshape(3 * E)
    ew_all = jnp.concatenate([edge_weight[None], cand_edge_weight],
                             axis=0).reshape(3 * E)
    ea_all = jnp.concatenate([edge_attr[None], cand_edge_attr],
                             axis=0).reshape(3 * E, DE)

    eenc = _edge_encode(ea_all, We, be)

    msg0 = _make_msg_call(N, (0, 0, 0))
    aggp0 = msg0(src_all, dst_all, ew_all, eenc, x)
    h1 = _dense_layer(0, x, aggp0, W1, b1, W2, b2, gamma, beta, eps)

    msg1 = _make_msg_call(G * N, (0, N, 2 * N))
    aggp1 = msg1(src_all, dst_all, ew_all, eenc, h1.reshape(G * N, H))
    h2 = _dense_layer(1, h1, aggp1, W1, b1, W2, b2, gamma, beta, eps)

    s, c = _pool(h2, batch.reshape(N, 1).astype(jnp.int32))
    return _head(s, c, Wm, bm, Wf, bf)
